# baseline (speedup 1.0000x reference)
import functools

import jax
import jax.numpy as jnp
from jax import lax
from jax.experimental import pallas as pl
from jax.experimental.pallas import tpu as pltpu
from jax.experimental.pallas import tpu_sc as plsc

_N_TILE = 2816
_N_BATCH = 2


def _sc_gather(table, flat_idx):
    num_idx = flat_idx.shape[0]
    embed_dim = table.shape[1]
    mesh = plsc.VectorSubcoreMesh(core_axis_name="c", subcore_axis_name="s")
    num_workers = mesh.num_cores * mesh.num_subcores
    b_per_w = num_idx // num_workers

    @functools.partial(
        pl.kernel,
        mesh=mesh,
        compiler_params=pltpu.CompilerParams(use_tc_tiling_on_sc=False),
        out_type=jax.ShapeDtypeStruct((num_idx, embed_dim), table.dtype),
        scratch_types=[
            pltpu.VMEM((b_per_w,), jnp.int32),
            pltpu.VMEM((b_per_w, embed_dim), table.dtype),
            pltpu.SemaphoreType.DMA,
        ],
    )
    def gather_kernel(table_hbm, idx_hbm, out_hbm, idx_v, rows_v, sem):
        wid = lax.axis_index("s") * mesh.num_cores + lax.axis_index("c")
        base = wid * b_per_w
        pltpu.sync_copy(idx_hbm.at[pl.ds(base, b_per_w)], idx_v)
        pltpu.async_copy(table_hbm.at[idx_v], rows_v, sem).wait()
        pltpu.sync_copy(rows_v, out_hbm.at[pl.ds(base, b_per_w)])

    return gather_kernel(table, flat_idx)


def _mm_body(f_ref, w_ref, b_ref, o_ref):
    j = pl.program_id(1)
    bb = o_ref.shape[1]
    f = f_ref[pl.ds(j * bb, bb), :].astype(jnp.bfloat16)
    w = w_ref[...].astype(jnp.bfloat16)
    acc = lax.dot_general(
        w, f, (((1,), (1,)), ((), ())), preferred_element_type=jnp.float32
    )
    o_ref[...] = acc + b_ref[...].T


def _tc_matmul(flatten, W, b):
    batch, k = flatten.shape
    vocab = W.shape[0]
    bb = batch // _N_BATCH
    grid = (pl.cdiv(vocab, _N_TILE), _N_BATCH)
    return pl.pallas_call(
        _mm_body,
        grid=grid,
        in_specs=[
            pl.BlockSpec((batch, k), lambda i, j: (0, 0)),
            pl.BlockSpec((_N_TILE, k), lambda i, j: (i, 0)),
            pl.BlockSpec((1, _N_TILE), lambda i, j: (0, i)),
        ],
        out_specs=pl.BlockSpec((_N_TILE, bb), lambda i, j: (i, j)),
        out_shape=jax.ShapeDtypeStruct((vocab, batch), jnp.float32),
        compiler_params=pltpu.CompilerParams(
            dimension_semantics=("parallel", "arbitrary"),
        ),
    )(flatten, W, b)


def kernel(x, table, W, b):
    batch, seq_len = x.shape
    embed_dim = table.shape[1]
    flat_idx = x.reshape(batch * seq_len).astype(jnp.int32)
    gathered = _sc_gather(table, flat_idx)
    flatten = gathered.reshape(batch, seq_len * embed_dim)
    out_t = _tc_matmul(flatten, W, b.reshape(1, -1))
    return out_t.T


# speedup vs baseline: 1.3488x; 1.3488x over previous
import functools

import jax
import jax.numpy as jnp
from jax import lax
from jax.experimental import pallas as pl
from jax.experimental.pallas import tpu as pltpu
from jax.experimental.pallas import tpu_sc as plsc

_N_TILE = 2048


def _sc_gather(table, flat_idx):
    num_idx = flat_idx.shape[0]
    embed_dim = table.shape[1]
    mesh = plsc.VectorSubcoreMesh(core_axis_name="c", subcore_axis_name="s")
    num_workers = mesh.num_cores * mesh.num_subcores
    b_per_w = num_idx // num_workers

    @functools.partial(
        pl.kernel,
        mesh=mesh,
        compiler_params=pltpu.CompilerParams(use_tc_tiling_on_sc=False),
        out_type=jax.ShapeDtypeStruct((num_idx, embed_dim), table.dtype),
        scratch_types=[
            pltpu.VMEM((b_per_w,), jnp.int32),
            pltpu.VMEM((b_per_w, embed_dim), table.dtype),
            pltpu.SemaphoreType.DMA,
        ],
    )
    def gather_kernel(table_hbm, idx_hbm, out_hbm, idx_v, rows_v, sem):
        wid = lax.axis_index("s") * mesh.num_cores + lax.axis_index("c")
        base = wid * b_per_w
        pltpu.sync_copy(idx_hbm.at[pl.ds(base, b_per_w)], idx_v)
        pltpu.async_copy(table_hbm.at[idx_v], rows_v, sem).wait()
        pltpu.sync_copy(rows_v, out_hbm.at[pl.ds(base, b_per_w)])

    return gather_kernel(table, flat_idx)


def _mm_body(f_ref, w_ref, b_ref, o_ref):
    f = f_ref[...].astype(jnp.bfloat16)
    w = w_ref[...].astype(jnp.bfloat16)
    acc = lax.dot_general(
        w, f, (((1,), (1,)), ((), ())), preferred_element_type=jnp.float32
    )
    o_ref[...] = acc + b_ref[...].T


def _tc_matmul(flatten, W, b):
    batch, k = flatten.shape
    vocab = W.shape[0]
    grid = (pl.cdiv(vocab, _N_TILE),)
    return pl.pallas_call(
        _mm_body,
        grid=grid,
        in_specs=[
            pl.BlockSpec((batch, k), lambda i: (0, 0)),
            pl.BlockSpec((_N_TILE, k), lambda i: (i, 0)),
            pl.BlockSpec((1, _N_TILE), lambda i: (0, i)),
        ],
        out_specs=pl.BlockSpec((_N_TILE, batch), lambda i: (i, 0)),
        out_shape=jax.ShapeDtypeStruct((vocab, batch), jnp.float32),
        compiler_params=pltpu.CompilerParams(
            dimension_semantics=("parallel",),
        ),
    )(flatten, W, b)


def kernel(x, table, W, b):
    batch, seq_len = x.shape
    embed_dim = table.shape[1]
    flat_idx = x.reshape(batch * seq_len).astype(jnp.int32)
    gathered = _sc_gather(table, flat_idx)
    flatten = gathered.reshape(batch, seq_len * embed_dim)
    out_t = _tc_matmul(flatten, W, b.reshape(1, -1))
    return out_t.T


# speedup vs baseline: 1.3888x; 1.0296x over previous
import functools

import jax
import jax.numpy as jnp
from jax import lax
from jax.experimental import pallas as pl
from jax.experimental.pallas import tpu as pltpu
from jax.experimental.pallas import tpu_sc as plsc

_N_TILE = 3072


def _sc_gather(table, flat_idx):
    num_idx = flat_idx.shape[0]
    embed_dim = table.shape[1]
    mesh = plsc.VectorSubcoreMesh(core_axis_name="c", subcore_axis_name="s")
    num_workers = mesh.num_cores * mesh.num_subcores
    b_per_w = num_idx // num_workers

    @functools.partial(
        pl.kernel,
        mesh=mesh,
        compiler_params=pltpu.CompilerParams(use_tc_tiling_on_sc=False),
        out_type=jax.ShapeDtypeStruct((num_idx, embed_dim), table.dtype),
        scratch_types=[
            pltpu.VMEM((b_per_w,), jnp.int32),
            pltpu.VMEM((b_per_w, embed_dim), table.dtype),
            pltpu.SemaphoreType.DMA,
        ],
    )
    def gather_kernel(table_hbm, idx_hbm, out_hbm, idx_v, rows_v, sem):
        wid = lax.axis_index("s") * mesh.num_cores + lax.axis_index("c")
        base = wid * b_per_w
        pltpu.sync_copy(idx_hbm.at[pl.ds(base, b_per_w)], idx_v)
        pltpu.async_copy(table_hbm.at[idx_v], rows_v, sem).wait()
        pltpu.sync_copy(rows_v, out_hbm.at[pl.ds(base, b_per_w)])

    return gather_kernel(table, flat_idx)


def _mm_body(f_ref, w_ref, b_ref, o_ref):
    f = f_ref[...].astype(jnp.bfloat16)
    w = w_ref[...].astype(jnp.bfloat16)
    acc = lax.dot_general(
        w, f, (((1,), (1,)), ((), ())), preferred_element_type=jnp.float32
    )
    o_ref[...] = acc + b_ref[...].T


def _tc_matmul(flatten, W, b):
    batch, k = flatten.shape
    vocab = W.shape[0]
    grid = (pl.cdiv(vocab, _N_TILE),)
    return pl.pallas_call(
        _mm_body,
        grid=grid,
        in_specs=[
            pl.BlockSpec((batch, k), lambda i: (0, 0)),
            pl.BlockSpec((_N_TILE, k), lambda i: (i, 0)),
            pl.BlockSpec((1, _N_TILE), lambda i: (0, i)),
        ],
        out_specs=pl.BlockSpec((_N_TILE, batch), lambda i: (i, 0)),
        out_shape=jax.ShapeDtypeStruct((vocab, batch), jnp.float32),
        compiler_params=pltpu.CompilerParams(
            dimension_semantics=("parallel",),
            vmem_limit_bytes=100 * 1024 * 1024,
        ),
    )(flatten, W, b)


def kernel(x, table, W, b):
    batch, seq_len = x.shape
    embed_dim = table.shape[1]
    flat_idx = x.reshape(batch * seq_len).astype(jnp.int32)
    gathered = _sc_gather(table, flat_idx)
    flatten = gathered.reshape(batch, seq_len * embed_dim)
    out_t = _tc_matmul(flatten, W, b.reshape(1, -1))
    return out_t.T


# speedup vs baseline: 1.3946x; 1.0042x over previous
import functools

import jax
import jax.numpy as jnp
from jax import lax
from jax.experimental import pallas as pl
from jax.experimental.pallas import tpu as pltpu
from jax.experimental.pallas import tpu_sc as plsc

_N_TILE = 3584


def _sc_gather(table, flat_idx):
    num_idx = flat_idx.shape[0]
    embed_dim = table.shape[1]
    mesh = plsc.VectorSubcoreMesh(core_axis_name="c", subcore_axis_name="s")
    num_workers = mesh.num_cores * mesh.num_subcores
    b_per_w = num_idx // num_workers

    @functools.partial(
        pl.kernel,
        mesh=mesh,
        compiler_params=pltpu.CompilerParams(use_tc_tiling_on_sc=False),
        out_type=jax.ShapeDtypeStruct((num_idx, embed_dim), table.dtype),
        scratch_types=[
            pltpu.VMEM((b_per_w,), jnp.int32),
            pltpu.VMEM((b_per_w, embed_dim), table.dtype),
            pltpu.SemaphoreType.DMA,
        ],
    )
    def gather_kernel(table_hbm, idx_hbm, out_hbm, idx_v, rows_v, sem):
        wid = lax.axis_index("s") * mesh.num_cores + lax.axis_index("c")
        base = wid * b_per_w
        pltpu.sync_copy(idx_hbm.at[pl.ds(base, b_per_w)], idx_v)
        pltpu.async_copy(table_hbm.at[idx_v], rows_v, sem).wait()
        pltpu.sync_copy(rows_v, out_hbm.at[pl.ds(base, b_per_w)])

    return gather_kernel(table, flat_idx)


def _mm_body(f_ref, w_ref, b_ref, o_ref):
    f = f_ref[...].astype(jnp.bfloat16)
    w = w_ref[...].astype(jnp.bfloat16)
    acc = lax.dot_general(
        w, f, (((1,), (1,)), ((), ())), preferred_element_type=jnp.float32
    )
    o_ref[...] = acc + b_ref[...].T


def _tc_matmul(flatten, W, b):
    batch, k = flatten.shape
    vocab = W.shape[0]
    grid = (pl.cdiv(vocab, _N_TILE),)
    return pl.pallas_call(
        _mm_body,
        grid=grid,
        in_specs=[
            pl.BlockSpec((batch, k), lambda i: (0, 0)),
            pl.BlockSpec((_N_TILE, k), lambda i: (i, 0)),
            pl.BlockSpec((1, _N_TILE), lambda i: (0, i)),
        ],
        out_specs=pl.BlockSpec((_N_TILE, batch), lambda i: (i, 0)),
        out_shape=jax.ShapeDtypeStruct((vocab, batch), jnp.float32),
        compiler_params=pltpu.CompilerParams(
            dimension_semantics=("parallel",),
            vmem_limit_bytes=100 * 1024 * 1024,
        ),
    )(flatten, W, b)


def kernel(x, table, W, b):
    batch, seq_len = x.shape
    embed_dim = table.shape[1]
    flat_idx = x.reshape(batch * seq_len).astype(jnp.int32)
    gathered = _sc_gather(table, flat_idx)
    flatten = gathered.reshape(batch, seq_len * embed_dim)
    out_t = _tc_matmul(flatten, W, b.reshape(1, -1))
    return out_t.T
